# Initial kernel scaffold; baseline (speedup 1.0000x reference)
import jax
import jax.numpy as jnp
from jax.experimental import pallas as pl


def kernel(unary, image, w_bilateral, w_spatial):
    raise NotImplementedError("write your pallas kernel here")


# speedup vs baseline: 1.0529x; 1.0529x over previous
import functools

import jax
import jax.numpy as jnp
from jax.experimental import pallas as pl
from jax.experimental.pallas import tpu as pltpu

H = 64
W_IMG = 64
C = 21
N = H * W_IMG
THETA_ALPHA = 80.0
THETA_BETA = 13.0
THETA_GAMMA = 3.0
N_ITERS = 5

BM_BUILD = 256
BM_ITER = 256


def _build_kernel(fbz_ref, fsz_ref, ft_ref, sqb_ref, sqs_ref, u_ref,
                  wb_ref, ws_ref, k_ref, q0_ref):
    j = pl.program_id(0)
    wb = wb_ref[0]
    ws = ws_ref[0]
    fbz = fbz_ref[...]
    fsz = fsz_ref[...]
    ft = ft_ref[...]
    gb = jnp.dot(fbz, ft, preferred_element_type=jnp.float32)
    gs = jnp.dot(fsz, ft, preferred_element_type=jnp.float32)
    sqb_rows = jnp.sum(fbz * fbz, axis=-1, keepdims=True)
    sqs_rows = jnp.sum(fsz * fsz, axis=-1, keepdims=True)
    d2b = jnp.maximum(sqb_rows + sqb_ref[...] - 2.0 * gb, 0.0)
    d2s = jnp.maximum(sqs_rows + sqs_ref[...] - 2.0 * gs, 0.0)
    k = wb * jnp.exp(-0.5 * d2b) + ws * jnp.exp(-0.5 * d2s)
    rows = j * BM_BUILD + jax.lax.broadcasted_iota(jnp.int32, (BM_BUILD, N), 0)
    cols = jax.lax.broadcasted_iota(jnp.int32, (BM_BUILD, N), 1)
    k_ref[...] = jnp.where(rows == cols, 0.0, k)
    logits = -u_ref[...]
    m = jnp.max(logits, axis=-1, keepdims=True)
    e = jnp.exp(logits - m)
    q0_ref[...] = e / jnp.sum(e, axis=-1, keepdims=True)


def _iter_kernel(k_ref, q_ref, u_ref, qo_ref):
    msg = jnp.dot(k_ref[...], q_ref[...], preferred_element_type=jnp.float32)
    pairwise = jnp.sum(msg, axis=-1, keepdims=True) - msg
    logits = -u_ref[...] - pairwise
    m = jnp.max(logits, axis=-1, keepdims=True)
    e = jnp.exp(logits - m)
    qo_ref[...] = e / jnp.sum(e, axis=-1, keepdims=True)


@functools.partial(jax.jit, static_argnames=())
def kernel(unary, image, w_bilateral, w_spatial):
    h, w, c = unary.shape
    n = h * w
    U = unary.reshape(n, c)

    ys, xs = jnp.meshgrid(jnp.arange(h, dtype=jnp.float32),
                          jnp.arange(w, dtype=jnp.float32), indexing="ij")
    coords = jnp.stack([xs, ys], axis=-1).reshape(-1, 2)
    rgb = image.reshape(-1, 3)
    fb = jnp.concatenate([coords / THETA_ALPHA, rgb / THETA_BETA], axis=-1)
    fs = coords / THETA_GAMMA
    zeros3 = jnp.zeros((n, 3), jnp.float32)
    zeros5 = jnp.zeros((n, 5), jnp.float32)
    fbz = jnp.concatenate([fb, zeros3], axis=-1)
    fsz = jnp.concatenate([zeros5, fs, zeros3[:, :1]], axis=-1)
    fall = fbz + fsz
    ft = fall.T
    sqb = jnp.sum(fb * fb, axis=-1)[None, :]
    sqs = jnp.sum(fs * fs, axis=-1)[None, :]
    wb = jnp.reshape(w_bilateral.astype(jnp.float32), (1,))
    ws = jnp.reshape(w_spatial.astype(jnp.float32), (1,))

    nb = n // BM_BUILD
    K, Q = pl.pallas_call(
        _build_kernel,
        grid=(nb,),
        in_specs=[
            pl.BlockSpec((BM_BUILD, 8), lambda j: (j, 0)),
            pl.BlockSpec((BM_BUILD, 8), lambda j: (j, 0)),
            pl.BlockSpec((8, n), lambda j: (0, 0)),
            pl.BlockSpec((1, n), lambda j: (0, 0)),
            pl.BlockSpec((1, n), lambda j: (0, 0)),
            pl.BlockSpec((BM_BUILD, c), lambda j: (j, 0)),
            pl.BlockSpec(memory_space=pltpu.SMEM),
            pl.BlockSpec(memory_space=pltpu.SMEM),
        ],
        out_specs=[
            pl.BlockSpec((BM_BUILD, n), lambda j: (j, 0)),
            pl.BlockSpec((BM_BUILD, c), lambda j: (j, 0)),
        ],
        out_shape=[
            jax.ShapeDtypeStruct((n, n), jnp.float32),
            jax.ShapeDtypeStruct((n, c), jnp.float32),
        ],
    )(fbz, fsz, ft, sqb, sqs, U, wb, ws)

    nbi = n // BM_ITER
    iter_call = pl.pallas_call(
        _iter_kernel,
        grid=(nbi,),
        in_specs=[
            pl.BlockSpec((BM_ITER, n), lambda j: (j, 0)),
            pl.BlockSpec((n, c), lambda j: (0, 0)),
            pl.BlockSpec((BM_ITER, c), lambda j: (j, 0)),
        ],
        out_specs=pl.BlockSpec((BM_ITER, c), lambda j: (j, 0)),
        out_shape=jax.ShapeDtypeStruct((n, c), jnp.float32),
    )
    for _ in range(N_ITERS):
        Q = iter_call(K, Q, U)
    return Q.reshape(h, w, c)


# speedup vs baseline: 1.3639x; 1.2954x over previous
import functools

import jax
import jax.numpy as jnp
from jax.experimental import pallas as pl
from jax.experimental.pallas import tpu as pltpu

H = 64
W_IMG = 64
C = 21
N = H * W_IMG
THETA_ALPHA = 80.0
THETA_BETA = 13.0
THETA_GAMMA = 3.0
N_ITERS = 5

BM = 256
NB = N // BM


def _affinity_tile(fz, ft, sq_cols, row0):
    g = jnp.dot(fz, ft, preferred_element_type=jnp.float32)
    sq_rows = jnp.sum(fz * fz, axis=-1, keepdims=True)
    d2 = jnp.maximum(sq_rows + sq_cols - 2.0 * g, 0.0)
    k = jnp.exp(-0.5 * d2)
    rows = row0 + jax.lax.broadcasted_iota(jnp.int32, (BM, N), 0)
    cols = jax.lax.broadcasted_iota(jnp.int32, (BM, N), 1)
    return jnp.where(rows == cols, 0.0, k)


def _ks_build_kernel(fsz_ref, ft_ref, sqs_ref, ks_ref):
    j = pl.program_id(0)
    ks = _affinity_tile(fsz_ref[...], ft_ref[...], sqs_ref[...], j * BM)
    ks_ref[...] = ks.astype(jnp.bfloat16)


def _fused_kernel(fbz_ref, ft_ref, sqb_ref, ut_ref, ks_ref,
                  wb_ref, ws_ref, out_ref, kb_s, qa_s, qb_s):
    t = pl.program_id(0)
    j = pl.program_id(1)

    @pl.when(t == 0)
    def _build():
        kb = _affinity_tile(fbz_ref[...], ft_ref[...], sqb_ref[...], j * BM)
        kb_s[pl.ds(j * BM, BM), :] = kb.astype(jnp.bfloat16)

        @pl.when(j == 0)
        def _init_q():
            logits = -ut_ref[...]
            m = jnp.max(logits, axis=0, keepdims=True)
            e = jnp.exp(logits - m)
            qa_s[...] = e / jnp.sum(e, axis=0, keepdims=True)

    def _step(src_s, dst_s):
        qtb = src_s[...].astype(jnp.bfloat16)
        kb_col = kb_s[:, pl.ds(j * BM, BM)]
        mb = jnp.dot(qtb, kb_col, preferred_element_type=jnp.float32)
        ms = jnp.dot(qtb, ks_ref[...], preferred_element_type=jnp.float32)
        msg = wb_ref[0] * mb + ws_ref[0] * ms
        pw = jnp.sum(msg, axis=0, keepdims=True) - msg
        logits = -ut_ref[:, pl.ds(j * BM, BM)] - pw
        m = jnp.max(logits, axis=0, keepdims=True)
        e = jnp.exp(logits - m)
        q = e / jnp.sum(e, axis=0, keepdims=True)
        dst_s[:, pl.ds(j * BM, BM)] = q

        @pl.when(t == N_ITERS)
        def _emit():
            out_ref[:, pl.ds(j * BM, BM)] = q

    @pl.when((t > 0) & (t % 2 == 1))
    def _odd():
        _step(qa_s, qb_s)

    @pl.when((t > 0) & (t % 2 == 0))
    def _even():
        _step(qb_s, qa_s)


@functools.partial(jax.jit, static_argnames=())
def kernel(unary, image, w_bilateral, w_spatial):
    h, w, c = unary.shape
    n = h * w
    UT = unary.reshape(n, c).T

    ys, xs = jnp.meshgrid(jnp.arange(h, dtype=jnp.float32),
                          jnp.arange(w, dtype=jnp.float32), indexing="ij")
    coords = jnp.stack([xs, ys], axis=-1).reshape(-1, 2)
    rgb = image.reshape(-1, 3)
    fb = jnp.concatenate([coords / THETA_ALPHA, rgb / THETA_BETA], axis=-1)
    fs = coords / THETA_GAMMA
    zeros3 = jnp.zeros((n, 3), jnp.float32)
    zeros5 = jnp.zeros((n, 5), jnp.float32)
    fbz = jnp.concatenate([fb, zeros3], axis=-1)
    fsz = jnp.concatenate([zeros5, fs, zeros3[:, :1]], axis=-1)
    ft = (fbz + fsz).T
    sqb = jnp.sum(fb * fb, axis=-1)[None, :]
    sqs = jnp.sum(fs * fs, axis=-1)[None, :]
    wb = jnp.reshape(w_bilateral.astype(jnp.float32), (1,))
    ws = jnp.reshape(w_spatial.astype(jnp.float32), (1,))

    Ks = pl.pallas_call(
        _ks_build_kernel,
        grid=(NB,),
        in_specs=[
            pl.BlockSpec((BM, 8), lambda j: (j, 0)),
            pl.BlockSpec((8, n), lambda j: (0, 0)),
            pl.BlockSpec((1, n), lambda j: (0, 0)),
        ],
        out_specs=pl.BlockSpec((BM, n), lambda j: (j, 0)),
        out_shape=jax.ShapeDtypeStruct((n, n), jnp.bfloat16),
    )(fsz, ft, sqs)

    QT = pl.pallas_call(
        _fused_kernel,
        grid=(N_ITERS + 1, NB),
        in_specs=[
            pl.BlockSpec((BM, 8), lambda t, j: (j, 0)),
            pl.BlockSpec((8, n), lambda t, j: (0, 0)),
            pl.BlockSpec((1, n), lambda t, j: (0, 0)),
            pl.BlockSpec((c, n), lambda t, j: (0, 0)),
            pl.BlockSpec((n, BM), lambda t, j: (0, jnp.where(t == 0, 0, j))),
            pl.BlockSpec(memory_space=pltpu.SMEM),
            pl.BlockSpec(memory_space=pltpu.SMEM),
        ],
        out_specs=pl.BlockSpec((c, n), lambda t, j: (0, 0)),
        out_shape=jax.ShapeDtypeStruct((c, n), jnp.float32),
        scratch_shapes=[
            pltpu.VMEM((n, n), jnp.bfloat16),
            pltpu.VMEM((c, n), jnp.float32),
            pltpu.VMEM((c, n), jnp.float32),
        ],
        compiler_params=pltpu.CompilerParams(
            vmem_limit_bytes=63 * 1024 * 1024,
        ),
    )(fbz, ft, sqb, UT, Ks, wb, ws)
    return QT.T.reshape(h, w, c)


# speedup vs baseline: 1.7314x; 1.2694x over previous
import functools

import jax
import jax.numpy as jnp
from jax.experimental import pallas as pl
from jax.experimental.pallas import tpu as pltpu

H = 64
W_IMG = 64
C = 21
N = H * W_IMG
THETA_ALPHA = 80.0
THETA_BETA = 13.0
THETA_GAMMA = 3.0
N_ITERS = 5

BM = 512
NB = N // BM
WIN_ROWS = 52
WIN = WIN_ROWS * W_IMG


def _win_start(j):
    start = jnp.clip(4 * j - 11, 0, (H - WIN_ROWS) // 2) * 128
    return pl.multiple_of(start, 128)


def _ks_build_kernel(fsz_ref, ft_ref, sqs_ref, ks_ref):
    j = pl.program_id(0)
    start = _win_start(j)
    fz = fsz_ref[pl.ds(start, WIN), :]
    ftc = ft_ref[:, pl.ds(j * BM, BM)]
    sq_cols = sqs_ref[:, pl.ds(j * BM, BM)]
    g = jnp.dot(fz, ftc, preferred_element_type=jnp.float32)
    sq_rows = jnp.sum(fz * fz, axis=-1, keepdims=True)
    d2 = jnp.maximum(sq_rows + sq_cols - 2.0 * g, 0.0)
    k = jnp.exp(-0.5 * d2)
    rows = start + jax.lax.broadcasted_iota(jnp.int32, (WIN, BM), 0)
    cols = j * BM + jax.lax.broadcasted_iota(jnp.int32, (WIN, BM), 1)
    ks_ref[...] = jnp.where(rows == cols, 0.0, k).astype(jnp.bfloat16)


def _fused_kernel(fbz_ref, ft_ref, sqb_ref, ut_ref, ks_ref,
                  wb_ref, ws_ref, out_ref, kb_s, qa_s, qb_s):
    t = pl.program_id(0)
    j = pl.program_id(1)

    @pl.when(t == 0)
    def _build():
        fz = fbz_ref[...]
        ft = ft_ref[...]
        g = jnp.dot(fz, ft, preferred_element_type=jnp.float32)
        sq_rows = jnp.sum(fz * fz, axis=-1, keepdims=True)
        d2 = jnp.maximum(sq_rows + sqb_ref[...] - 2.0 * g, 0.0)
        kb = jnp.exp(-0.5 * d2)
        rows = j * BM + jax.lax.broadcasted_iota(jnp.int32, (BM, N), 0)
        cols = jax.lax.broadcasted_iota(jnp.int32, (BM, N), 1)
        kb = jnp.where(rows == cols, 0.0, kb)
        kb_s[pl.ds(j * BM, BM), :] = kb.astype(jnp.bfloat16)

        @pl.when(j == 0)
        def _init_q():
            logits = -ut_ref[...]
            m = jnp.max(logits, axis=0, keepdims=True)
            e = jnp.exp(logits - m)
            qa_s[...] = e / jnp.sum(e, axis=0, keepdims=True)

    def _step(src_s, dst_s):
        qtb = src_s[...].astype(jnp.bfloat16)
        qwin = src_s[:, pl.ds(_win_start(j), WIN)].astype(jnp.bfloat16)
        kb_col = kb_s[:, pl.ds(j * BM, BM)]
        mb = jnp.dot(qtb, kb_col, preferred_element_type=jnp.float32)
        ms = jnp.dot(qwin, ks_ref[...], preferred_element_type=jnp.float32)
        msg = wb_ref[0] * mb + ws_ref[0] * ms
        pw = jnp.sum(msg, axis=0, keepdims=True) - msg
        logits = -ut_ref[:, pl.ds(j * BM, BM)] - pw
        m = jnp.max(logits, axis=0, keepdims=True)
        e = jnp.exp(logits - m)
        q = e / jnp.sum(e, axis=0, keepdims=True)
        dst_s[:, pl.ds(j * BM, BM)] = q

        @pl.when(t == N_ITERS)
        def _emit():
            out_ref[:, pl.ds(j * BM, BM)] = q

    @pl.when((t > 0) & (t % 2 == 1))
    def _odd():
        _step(qa_s, qb_s)

    @pl.when((t > 0) & (t % 2 == 0))
    def _even():
        _step(qb_s, qa_s)


@functools.partial(jax.jit, static_argnames=())
def kernel(unary, image, w_bilateral, w_spatial):
    h, w, c = unary.shape
    n = h * w
    UT = unary.reshape(n, c).T

    ys, xs = jnp.meshgrid(jnp.arange(h, dtype=jnp.float32),
                          jnp.arange(w, dtype=jnp.float32), indexing="ij")
    coords = jnp.stack([xs, ys], axis=-1).reshape(-1, 2)
    rgb = image.reshape(-1, 3)
    fb = jnp.concatenate([coords / THETA_ALPHA, rgb / THETA_BETA], axis=-1)
    fs = coords / THETA_GAMMA
    zeros3 = jnp.zeros((n, 3), jnp.float32)
    zeros5 = jnp.zeros((n, 5), jnp.float32)
    fbz = jnp.concatenate([fb, zeros3], axis=-1)
    fsz = jnp.concatenate([zeros5, fs, zeros3[:, :1]], axis=-1)
    ft = (fbz + fsz).T
    sqb = jnp.sum(fb * fb, axis=-1)[None, :]
    sqs = jnp.sum(fs * fs, axis=-1)[None, :]
    wb = jnp.reshape(w_bilateral.astype(jnp.float32), (1,))
    ws = jnp.reshape(w_spatial.astype(jnp.float32), (1,))

    Ks = pl.pallas_call(
        _ks_build_kernel,
        grid=(NB,),
        in_specs=[
            pl.BlockSpec((n, 8), lambda j: (0, 0)),
            pl.BlockSpec((8, n), lambda j: (0, 0)),
            pl.BlockSpec((1, n), lambda j: (0, 0)),
        ],
        out_specs=pl.BlockSpec((WIN, BM), lambda j: (0, j)),
        out_shape=jax.ShapeDtypeStruct((WIN, n), jnp.bfloat16),
    )(fsz, ft, sqs)

    QT = pl.pallas_call(
        _fused_kernel,
        grid=(N_ITERS + 1, NB),
        in_specs=[
            pl.BlockSpec((BM, 8), lambda t, j: (j, 0)),
            pl.BlockSpec((8, n), lambda t, j: (0, 0)),
            pl.BlockSpec((1, n), lambda t, j: (0, 0)),
            pl.BlockSpec((c, n), lambda t, j: (0, 0)),
            pl.BlockSpec((WIN, BM), lambda t, j: (0, jnp.where(t == 0, 0, j))),
            pl.BlockSpec(memory_space=pltpu.SMEM),
            pl.BlockSpec(memory_space=pltpu.SMEM),
        ],
        out_specs=pl.BlockSpec((c, n), lambda t, j: (0, 0)),
        out_shape=jax.ShapeDtypeStruct((c, n), jnp.float32),
        scratch_shapes=[
            pltpu.VMEM((n, n), jnp.bfloat16),
            pltpu.VMEM((c, n), jnp.float32),
            pltpu.VMEM((c, n), jnp.float32),
        ],
        compiler_params=pltpu.CompilerParams(
            vmem_limit_bytes=63 * 1024 * 1024,
        ),
    )(fbz, ft, sqb, UT, Ks, wb, ws)
    return QT.T.reshape(h, w, c)


# speedup vs baseline: 1.8353x; 1.0600x over previous
import functools

import jax
import jax.numpy as jnp
from jax.experimental import pallas as pl
from jax.experimental.pallas import tpu as pltpu

H = 64
W_IMG = 64
C = 21
N = H * W_IMG
THETA_ALPHA = 80.0
THETA_BETA = 13.0
THETA_GAMMA = 3.0
N_ITERS = 5

BM = 512
NB = N // BM
WIN_ROWS = 44
WIN = WIN_ROWS * W_IMG


def _win_start(j):
    start = jnp.clip(4 * j - 9, 0, (H - WIN_ROWS) // 2) * 128
    return pl.multiple_of(start, 128)


def _ks_build_kernel(fsz_ref, ft_ref, sqs_ref, ks_ref):
    j = pl.program_id(0)
    start = _win_start(j)
    fz = fsz_ref[pl.ds(start, WIN), :]
    ftc = ft_ref[:, pl.ds(j * BM, BM)]
    sq_cols = sqs_ref[:, pl.ds(j * BM, BM)]
    g = jnp.dot(fz, ftc, preferred_element_type=jnp.float32)
    sq_rows = jnp.sum(fz * fz, axis=-1, keepdims=True)
    d2 = jnp.maximum(sq_rows + sq_cols - 2.0 * g, 0.0)
    k = jnp.exp(-0.5 * d2)
    rows = start + jax.lax.broadcasted_iota(jnp.int32, (WIN, BM), 0)
    cols = j * BM + jax.lax.broadcasted_iota(jnp.int32, (WIN, BM), 1)
    ks_ref[...] = jnp.where(rows == cols, 0.0, k).astype(jnp.bfloat16)


def _fused_kernel(fbz_ref, ft_ref, sqb_ref, ut_ref, ks_ref,
                  wb_ref, ws_ref, out_ref, kb_s, qa_s, qb_s):
    t = pl.program_id(0)
    j = pl.program_id(1)

    @pl.when(t == 0)
    def _build():
        fz = fbz_ref[...]
        ft = ft_ref[...]
        g = jnp.dot(fz, ft, preferred_element_type=jnp.float32)
        sq_rows = jnp.sum(fz * fz, axis=-1, keepdims=True)
        d2 = jnp.maximum(sq_rows + sqb_ref[...] - 2.0 * g, 0.0)
        kb = jnp.exp(-0.5 * d2)
        rows = j * BM + jax.lax.broadcasted_iota(jnp.int32, (BM, N), 0)
        cols = jax.lax.broadcasted_iota(jnp.int32, (BM, N), 1)
        kb = jnp.where(rows == cols, 0.0, kb)
        kb_s[pl.ds(j * BM, BM), :] = kb.astype(jnp.bfloat16)

        @pl.when(j == 0)
        def _init_q():
            logits = -ut_ref[...]
            m = jnp.max(logits, axis=0, keepdims=True)
            e = jnp.exp(logits - m)
            q0 = e / jnp.sum(e, axis=0, keepdims=True)
            qa_s[...] = q0.astype(jnp.bfloat16)

    def _step(src_s, dst_s):
        qtb = src_s[...]
        qwin = src_s[:, pl.ds(_win_start(j), WIN)]
        kb_col = kb_s[:, pl.ds(j * BM, BM)]
        mb = jnp.dot(qtb, kb_col, preferred_element_type=jnp.float32)
        ms = jnp.dot(qwin, ks_ref[...], preferred_element_type=jnp.float32)
        msg = wb_ref[0] * mb + ws_ref[0] * ms
        pw = jnp.sum(msg, axis=0, keepdims=True) - msg
        logits = -ut_ref[:, pl.ds(j * BM, BM)] - pw
        m = jnp.max(logits, axis=0, keepdims=True)
        e = jnp.exp(logits - m)
        q = e / jnp.sum(e, axis=0, keepdims=True)
        dst_s[:, pl.ds(j * BM, BM)] = q.astype(jnp.bfloat16)

        @pl.when(t == N_ITERS)
        def _emit():
            out_ref[:, pl.ds(j * BM, BM)] = q

    @pl.when((t > 0) & (t % 2 == 1))
    def _odd():
        _step(qa_s, qb_s)

    @pl.when((t > 0) & (t % 2 == 0))
    def _even():
        _step(qb_s, qa_s)


@functools.partial(jax.jit, static_argnames=())
def kernel(unary, image, w_bilateral, w_spatial):
    h, w, c = unary.shape
    n = h * w
    UT = unary.reshape(n, c).T

    ys, xs = jnp.meshgrid(jnp.arange(h, dtype=jnp.float32),
                          jnp.arange(w, dtype=jnp.float32), indexing="ij")
    coords = jnp.stack([xs, ys], axis=-1).reshape(-1, 2)
    rgb = image.reshape(-1, 3)
    fb = jnp.concatenate([coords / THETA_ALPHA, rgb / THETA_BETA], axis=-1)
    fs = coords / THETA_GAMMA
    zeros3 = jnp.zeros((n, 3), jnp.float32)
    zeros5 = jnp.zeros((n, 5), jnp.float32)
    fbz = jnp.concatenate([fb, zeros3], axis=-1)
    fsz = jnp.concatenate([zeros5, fs, zeros3[:, :1]], axis=-1)
    ft = (fbz + fsz).T
    sqb = jnp.sum(fb * fb, axis=-1)[None, :]
    sqs = jnp.sum(fs * fs, axis=-1)[None, :]
    wb = jnp.reshape(w_bilateral.astype(jnp.float32), (1,))
    ws = jnp.reshape(w_spatial.astype(jnp.float32), (1,))

    Ks = pl.pallas_call(
        _ks_build_kernel,
        grid=(NB,),
        in_specs=[
            pl.BlockSpec((n, 8), lambda j: (0, 0)),
            pl.BlockSpec((8, n), lambda j: (0, 0)),
            pl.BlockSpec((1, n), lambda j: (0, 0)),
        ],
        out_specs=pl.BlockSpec((WIN, BM), lambda j: (0, j)),
        out_shape=jax.ShapeDtypeStruct((WIN, n), jnp.bfloat16),
    )(fsz, ft, sqs)

    QT = pl.pallas_call(
        _fused_kernel,
        grid=(N_ITERS + 1, NB),
        in_specs=[
            pl.BlockSpec((BM, 8), lambda t, j: (j, 0)),
            pl.BlockSpec((8, n), lambda t, j: (0, 0)),
            pl.BlockSpec((1, n), lambda t, j: (0, 0)),
            pl.BlockSpec((c, n), lambda t, j: (0, 0)),
            pl.BlockSpec((WIN, BM), lambda t, j: (0, jnp.where(t == 0, 0, j))),
            pl.BlockSpec(memory_space=pltpu.SMEM),
            pl.BlockSpec(memory_space=pltpu.SMEM),
        ],
        out_specs=pl.BlockSpec((c, n), lambda t, j: (0, 0)),
        out_shape=jax.ShapeDtypeStruct((c, n), jnp.float32),
        scratch_shapes=[
            pltpu.VMEM((n, n), jnp.bfloat16),
            pltpu.VMEM((c, n), jnp.bfloat16),
            pltpu.VMEM((c, n), jnp.bfloat16),
        ],
        compiler_params=pltpu.CompilerParams(
            vmem_limit_bytes=63 * 1024 * 1024,
        ),
    )(fbz, ft, sqb, UT, Ks, wb, ws)
    return QT.T.reshape(h, w, c)
